# baseline (speedup 1.0000x reference)
import jax
import jax.numpy as jnp
import numpy as np
from jax.experimental import pallas as pl
from jax.experimental.pallas import tpu as pltpu

_NPAD = 100352
_NROW = 784


def _update_body(dt_ref, px, py, pz, gx, gy, gz, ox, oy, oz, *, n_atoms, n_mov):
    dt = dt_ref[0]
    def f(g):
        v = g[...]
        return jnp.where(jnp.isnan(v), 0.0, v)

    vx, vy, vz = f(gx), f(gy), f(gz)
    fn = jnp.sqrt(vx * vx + vy * vy + vz * vz)
    thresh = 0.1 / dt
    scale = jnp.where(fn > thresh, thresh / (fn + 1e-12), 1.0)
    vx, vy, vz = vx * scale, vy * scale, vz * scale
    row = jax.lax.broadcasted_iota(jnp.int32, (_NROW, 128), 0)
    col = jax.lax.broadcasted_iota(jnp.int32, (_NROW, 128), 1)
    flat = row * 128 + col
    mov = (flat < n_mov).astype(jnp.float32) * dt
    ox[...] = px[...] + vx * mov
    oy[...] = py[...] + vy * mov
    oz[...] = pz[...] + vz * mov


def _final_update(pos, grad, n_mov, dt):
    n = pos.shape[0]
    pad = _NPAD - n
    posp = jnp.pad(pos, ((0, pad), (0, 0))).T.reshape(3, _NROW, 128)
    gradp = jnp.pad(grad, ((0, pad), (0, 0))).T.reshape(3, _NROW, 128)
    import functools
    body = functools.partial(_update_body, n_atoms=n, n_mov=n_mov)
    outs = pl.pallas_call(
        body,
        in_specs=[pl.BlockSpec(memory_space=pltpu.SMEM)] + [pl.BlockSpec()] * 6,
        out_shape=[jax.ShapeDtypeStruct((_NROW, 128), jnp.float32)] * 3,
    )(dt.reshape(1), posp[0], posp[1], posp[2], gradp[0], gradp[1], gradp[2])
    out = jnp.stack(outs).reshape(3, _NPAD).T[:n]
    return out


def kernel(pos, bond_idcs, bond_eq_val, bond_tolerance, angle_idcs,
           angle_eq_val, angle_tolerance, dih_idcs, dih_eq_val,
           movable_pos_idcs, dtau):
    dt = dtau[0]
    grad = jnp.zeros_like(pos)

    Nb = bond_idcs.shape[0]
    p0 = pos[bond_idcs[:, 0]]
    p1 = pos[bond_idcs[:, 1]]
    vec = p1 - p0
    r = jnp.sqrt(jnp.sum(vec * vec, axis=-1))
    u = vec / r[:, None]
    dr = r - bond_eq_val
    act = (dr * dr - bond_tolerance * bond_tolerance) > 0
    coef = (1000.0 / Nb) * 2.0 * dr * act.astype(jnp.float32)
    c1 = coef[:, None] * u
    grad = grad.at[bond_idcs[:, 1]].add(c1)
    grad = grad.at[bond_idcs[:, 0]].add(-c1)

    Na = angle_idcs.shape[0]
    q0 = pos[angle_idcs[:, 0]]
    q1 = pos[angle_idcs[:, 1]]
    q2 = pos[angle_idcs[:, 2]]
    b0 = q0 - q1
    b1 = q2 - q1
    n0 = jnp.sqrt(jnp.sum(b0 * b0, axis=-1))
    n1 = jnp.sqrt(jnp.sum(b1 * b1, axis=-1))
    d = jnp.sum(b0 * b1, axis=-1)
    q = n0 * n1 + 1e-12
    c = d / q
    lo, hi = -1.0 + 1e-7, 1.0 - 1e-7
    c_cl = jnp.clip(c, lo, hi)
    theta = jnp.arccos(c_cl)
    dth = theta - angle_eq_val
    act_a = (dth * dth - angle_tolerance * angle_tolerance) > 0
    gtheta = (150.0 / Na) * 2.0 * dth * act_a.astype(jnp.float32)
    inside = ((c > lo) & (c < hi)).astype(jnp.float32)
    gc = gtheta * (-1.0 / jnp.sqrt(1.0 - c_cl * c_cl)) * inside
    gd = gc / q
    gq = -gc * d / (q * q)
    gb0 = gd[:, None] * b1 + (gq * n1)[:, None] * (b0 / n0[:, None])
    gb1 = gd[:, None] * b0 + (gq * n0)[:, None] * (b1 / n1[:, None])
    grad = grad.at[angle_idcs[:, 0]].add(gb0)
    grad = grad.at[angle_idcs[:, 2]].add(gb1)
    grad = grad.at[angle_idcs[:, 1]].add(-(gb0 + gb1))

    Nd = dih_idcs.shape[0]
    s0 = pos[dih_idcs[:, 0]]
    s1 = pos[dih_idcs[:, 1]]
    s2 = pos[dih_idcs[:, 2]]
    s3 = pos[dih_idcs[:, 3]]
    b0d = s0 - s1
    b1d = s2 - s1
    b2d = s3 - s2
    n1d = jnp.sqrt(jnp.sum(b1d * b1d, axis=-1, keepdims=True))
    b1n = b1d / (n1d + 1e-12)
    sv = jnp.sum(b0d * b1n, axis=-1, keepdims=True)
    v = b0d - sv * b1n
    sw = jnp.sum(b2d * b1n, axis=-1, keepdims=True)
    w = b2d - sw * b1n
    cr = jnp.cross(b1n, v)
    x = jnp.sum(v * w, axis=-1)
    y = jnp.sum(cr * w, axis=-1)
    phi = jnp.arctan2(y, x)
    dlt = phi - dih_eq_val
    gphi = (1.0 / Nd) * (-jnp.sin(dlt - np.pi) + jnp.cos(dlt - np.pi / 2))
    den = x * x + y * y
    gx = -y / den * gphi
    gy = x / den * gphi
    gv = gx[:, None] * w
    gw = gx[:, None] * v + gy[:, None] * cr
    gcr = gy[:, None] * w
    gb1n = jnp.cross(v, gcr)
    gv = gv + jnp.cross(gcr, b1n)
    gb2 = gw
    gsw = -jnp.sum(gw * b1n, axis=-1, keepdims=True)
    gb1n = gb1n - sw * gw
    gb2 = gb2 + gsw * b1n
    gb1n = gb1n + gsw * b2d
    gb0d = gv
    gsv = -jnp.sum(gv * b1n, axis=-1, keepdims=True)
    gb1n = gb1n - sv * gv
    gb0d = gb0d + gsv * b1n
    gb1n = gb1n + gsv * b0d
    gb1d = gb1n / (n1d + 1e-12)
    gn1 = -jnp.sum(gb1n * b1d, axis=-1, keepdims=True) / (n1d + 1e-12) ** 2
    gb1d = gb1d + gn1 * (b1d / n1d)
    grad = grad.at[dih_idcs[:, 0]].add(gb0d)
    grad = grad.at[dih_idcs[:, 1]].add(-gb0d - gb1d)
    grad = grad.at[dih_idcs[:, 2]].add(gb1d - gb2)
    grad = grad.at[dih_idcs[:, 3]].add(gb2)

    n_mov = movable_pos_idcs.shape[0]
    return _final_update(pos, -grad, n_mov, dt)


# speedup vs baseline: 13.8566x; 13.8566x over previous
import functools

import jax
import jax.numpy as jnp
import numpy as np
from jax import lax
from jax.experimental import pallas as pl
from jax.experimental.pallas import tpu as pltpu
from jax.experimental.pallas import tpu_sc as plsc

NB, NA, ND = 100000, 200000, 300000
N_ATOM = 100000
NBP, NAP, NDP = 106496, 204800, 303104
NPOS = 100352
DUMMY = NPOS - 1
TOT = 2 * NBP + 3 * NAP + 4 * NDP
NW = 32
PER_W = TOT // NW
CHUNK = 3984
NCHUNK = PER_W // CHUNK
STRIPE = NPOS // 16

@functools.cache
def _sc_kernels():
    mesh = plsc.VectorSubcoreMesh(core_axis_name="c", subcore_axis_name="s")
    cparams = pltpu.CompilerParams(use_tc_tiling_on_sc=False)
    f32 = jnp.float32

    @functools.partial(
        pl.kernel, mesh=mesh, compiler_params=cparams,
        out_type=[jax.ShapeDtypeStruct((TOT,), f32)] * 3,
        scratch_types=[
            pltpu.VMEM((CHUNK,), jnp.int32),
            pltpu.VMEM((CHUNK,), f32),
            pltpu.VMEM((CHUNK,), f32),
            pltpu.VMEM((CHUNK,), f32),
            pltpu.SemaphoreType.DMA,
        ],
    )
    def sc_gather(px_hbm, py_hbm, pz_hbm, idx_hbm, ox_hbm, oy_hbm, oz_hbm,
                  idx_v, xv, yv, zv, sem):
        wid = lax.axis_index("s") * 2 + lax.axis_index("c")
        for j in range(NCHUNK):
            base = wid * PER_W + j * CHUNK
            sl = pl.ds(base, CHUNK)
            pltpu.sync_copy(idx_hbm.at[sl], idx_v)
            cx = pltpu.async_copy(px_hbm.at[idx_v], xv, sem)
            cy = pltpu.async_copy(py_hbm.at[idx_v], yv, sem)
            cz = pltpu.async_copy(pz_hbm.at[idx_v], zv, sem)
            cx.wait()
            cy.wait()
            cz.wait()
            pltpu.sync_copy(xv, ox_hbm.at[sl])
            pltpu.sync_copy(yv, oy_hbm.at[sl])
            pltpu.sync_copy(zv, oz_hbm.at[sl])

    @functools.partial(
        pl.kernel, mesh=mesh, compiler_params=cparams,
        out_type=[jax.ShapeDtypeStruct((2 * NPOS,), f32)] * 3,
        scratch_types=[
            pltpu.VMEM((CHUNK,), jnp.int32),
            pltpu.VMEM((CHUNK,), f32),
            pltpu.VMEM((CHUNK,), f32),
            pltpu.VMEM((CHUNK,), f32),
            pltpu.VMEM_SHARED((NPOS,), f32),
            pltpu.VMEM_SHARED((NPOS,), f32),
            pltpu.VMEM_SHARED((NPOS,), f32),
        ],
    )
    def sc_scatter(cx_hbm, cy_hbm, cz_hbm, idx_hbm, zeros_hbm,
                   ox_hbm, oy_hbm, oz_hbm,
                   idx_v, xv, yv, zv, gx_sh, gy_sh, gz_sh):
        cid = lax.axis_index("c")
        sid = lax.axis_index("s")
        wid = sid * 2 + cid
        stripe = pl.ds(sid * STRIPE, STRIPE)
        pltpu.sync_copy(zeros_hbm, gx_sh.at[stripe])
        pltpu.sync_copy(zeros_hbm, gy_sh.at[stripe])
        pltpu.sync_copy(zeros_hbm, gz_sh.at[stripe])
        plsc.subcore_barrier()
        for j in range(NCHUNK):
            base = wid * PER_W + j * CHUNK
            sl = pl.ds(base, CHUNK)
            pltpu.sync_copy(idx_hbm.at[sl], idx_v)
            pltpu.sync_copy(cx_hbm.at[sl], xv)
            pltpu.sync_copy(cy_hbm.at[sl], yv)
            pltpu.sync_copy(cz_hbm.at[sl], zv)
            pltpu.sync_copy(xv, gx_sh.at[idx_v], add=True)
            pltpu.sync_copy(yv, gy_sh.at[idx_v], add=True)
            pltpu.sync_copy(zv, gz_sh.at[idx_v], add=True)
        plsc.subcore_barrier()
        out_off = pl.ds(cid * NPOS + sid * STRIPE, STRIPE)
        pltpu.sync_copy(gx_sh.at[stripe], ox_hbm.at[out_off])
        pltpu.sync_copy(gy_sh.at[stripe], oy_hbm.at[out_off])
        pltpu.sync_copy(gz_sh.at[stripe], oz_hbm.at[out_off])

    return sc_gather, sc_scatter


def _sin_poly(t):
    t2 = t * t
    return t * (1.0 + t2 * (-1.0 / 6 + t2 * (1.0 / 120 + t2 * (-1.0 / 5040 + t2 / 362880))))


def _cos_poly(t):
    t2 = t * t
    return 1.0 + t2 * (-0.5 + t2 * (1.0 / 24 + t2 * (-1.0 / 720 + t2 * (1.0 / 40320 - t2 / 3628800))))


def _arccos_poly(c):
    t = jnp.abs(c)
    s = jnp.sqrt(1.0 - t)
    p = 1.5707288 + t * (-0.2121144 + t * (0.0742610 - 0.0187293 * t))
    r = s * p
    return jnp.where(c >= 0, r, np.pi - r)


def _bond_body(eq, tol, x0, y0, z0, x1, y1, z1, ox0, oy0, oz0, ox1, oy1, oz1):
    vx = x1[...] - x0[...]
    vy = y1[...] - y0[...]
    vz = z1[...] - z0[...]
    r = jnp.sqrt(vx * vx + vy * vy + vz * vz)
    ir = 1.0 / r
    dr = r - eq[...]
    t = tol[...]
    act = ((dr * dr - t * t) > 0).astype(jnp.float32)
    coef = (2000.0 / NB) * dr * act * ir
    ox1[...] = coef * vx
    oy1[...] = coef * vy
    oz1[...] = coef * vz
    ox0[...] = -coef * vx
    oy0[...] = -coef * vy
    oz0[...] = -coef * vz


def _angle_body(eq, tol, x0, y0, z0, x1, y1, z1, x2, y2, z2,
                o0x, o0y, o0z, o1x, o1y, o1z, o2x, o2y, o2z):
    b0x = x0[...] - x1[...]
    b0y = y0[...] - y1[...]
    b0z = z0[...] - z1[...]
    b1x = x2[...] - x1[...]
    b1y = y2[...] - y1[...]
    b1z = z2[...] - z1[...]
    n0 = jnp.sqrt(b0x * b0x + b0y * b0y + b0z * b0z)
    n1 = jnp.sqrt(b1x * b1x + b1y * b1y + b1z * b1z)
    d = b0x * b1x + b0y * b1y + b0z * b1z
    q = n0 * n1 + 1e-12
    c = d / q
    lo, hi = -1.0 + 1e-7, 1.0 - 1e-7
    c_cl = jnp.clip(c, lo, hi)
    theta = _arccos_poly(c_cl)
    dth = theta - eq[...]
    t = tol[...]
    act = ((dth * dth - t * t) > 0).astype(jnp.float32)
    gtheta = (300.0 / NA) * dth * act
    inside = ((c > lo) & (c < hi)).astype(jnp.float32)
    gc = -gtheta * lax.rsqrt(1.0 - c_cl * c_cl) * inside
    gd = gc / q
    gq = -gc * d / (q * q)
    f0 = gq * n1 / n0
    f1 = gq * n0 / n1
    g0x = gd * b1x + f0 * b0x
    g0y = gd * b1y + f0 * b0y
    g0z = gd * b1z + f0 * b0z
    g1x = gd * b0x + f1 * b1x
    g1y = gd * b0y + f1 * b1y
    g1z = gd * b0z + f1 * b1z
    o0x[...] = g0x
    o0y[...] = g0y
    o0z[...] = g0z
    o2x[...] = g1x
    o2y[...] = g1y
    o2z[...] = g1z
    o1x[...] = -(g0x + g1x)
    o1y[...] = -(g0y + g1y)
    o1z[...] = -(g0z + g1z)


def _dih_body(eq, x0, y0, z0, x1, y1, z1, x2, y2, z2, x3, y3, z3,
              o0x, o0y, o0z, o1x, o1y, o1z, o2x, o2y, o2z, o3x, o3y, o3z):
    b0x = x0[...] - x1[...]
    b0y = y0[...] - y1[...]
    b0z = z0[...] - z1[...]
    b1x = x2[...] - x1[...]
    b1y = y2[...] - y1[...]
    b1z = z2[...] - z1[...]
    b2x = x3[...] - x2[...]
    b2y = y3[...] - y2[...]
    b2z = z3[...] - z2[...]
    n1 = jnp.sqrt(b1x * b1x + b1y * b1y + b1z * b1z)
    inb = 1.0 / (n1 + 1e-12)
    ux, uy, uz = b1x * inb, b1y * inb, b1z * inb
    sv = b0x * ux + b0y * uy + b0z * uz
    vx_, vy_, vz_ = b0x - sv * ux, b0y - sv * uy, b0z - sv * uz
    sw = b2x * ux + b2y * uy + b2z * uz
    wx, wy, wz = b2x - sw * ux, b2y - sw * uy, b2z - sw * uz
    crx = uy * vz_ - uz * vy_
    cry = uz * vx_ - ux * vz_
    crz = ux * vy_ - uy * vx_
    x = vx_ * wx + vy_ * wy + vz_ * wz
    y = crx * wx + cry * wy + crz * wz
    den = x * x + y * y
    iden = 1.0 / den
    irho = lax.rsqrt(den)
    sphi = y * irho
    cphi = x * irho
    e = eq[...]
    seq = _sin_poly(e)
    ceq = _cos_poly(e)
    sdlt = sphi * ceq - cphi * seq
    gphi = (2.0 / ND) * sdlt
    gx = -y * iden * gphi
    gy = x * iden * gphi
    gvx, gvy, gvz = gx * wx, gx * wy, gx * wz
    gwx = gx * vx_ + gy * crx
    gwy = gx * vy_ + gy * cry
    gwz = gx * vz_ + gy * crz
    gcrx, gcry, gcrz = gy * wx, gy * wy, gy * wz
    gux = vy_ * gcrz - vz_ * gcry
    guy = vz_ * gcrx - vx_ * gcrz
    guz = vx_ * gcry - vy_ * gcrx
    gvx += gcry * uz - gcrz * uy
    gvy += gcrz * ux - gcrx * uz
    gvz += gcrx * uy - gcry * ux
    gb2x, gb2y, gb2z = gwx, gwy, gwz
    gsw = -(gwx * ux + gwy * uy + gwz * uz)
    gux -= sw * gwx
    guy -= sw * gwy
    guz -= sw * gwz
    gb2x += gsw * ux
    gb2y += gsw * uy
    gb2z += gsw * uz
    gux += gsw * b2x
    guy += gsw * b2y
    guz += gsw * b2z
    gb0x, gb0y, gb0z = gvx, gvy, gvz
    gsv = -(gvx * ux + gvy * uy + gvz * uz)
    gux -= sv * gvx
    guy -= sv * gvy
    guz -= sv * gvz
    gb0x += gsv * ux
    gb0y += gsv * uy
    gb0z += gsv * uz
    gux += gsv * b0x
    guy += gsv * b0y
    guz += gsv * b0z
    gb1x = gux * inb
    gb1y = guy * inb
    gb1z = guz * inb
    gn1 = -(gux * b1x + guy * b1y + guz * b1z) * inb * inb
    fin = gn1 / n1
    gb1x += fin * b1x
    gb1y += fin * b1y
    gb1z += fin * b1z
    o0x[...] = gb0x
    o0y[...] = gb0y
    o0z[...] = gb0z
    o1x[...] = -gb0x - gb1x
    o1y[...] = -gb0y - gb1y
    o1z[...] = -gb0z - gb1z
    o2x[...] = gb1x - gb2x
    o2y[...] = gb1y - gb2y
    o2z[...] = gb1z - gb2z
    o3x[...] = gb2x
    o3y[...] = gb2y
    o3z[...] = gb2z


def _update_body(dt_ref, px, py, pz, ax, ay, az, bx, by, bz, ox, oy, oz):
    dt = dt_ref[0]

    def clean(v):
        return jnp.where(jnp.isnan(v), 0.0, v)

    gx = clean(-(ax[...] + bx[...]))
    gy = clean(-(ay[...] + by[...]))
    gz = clean(-(az[...] + bz[...]))
    fn = jnp.sqrt(gx * gx + gy * gy + gz * gz)
    thresh = 0.1 / dt
    scale = jnp.where(fn > thresh, thresh / (fn + 1e-12), 1.0)
    row = lax.broadcasted_iota(jnp.int32, (98, 1024), 0)
    col = lax.broadcasted_iota(jnp.int32, (98, 1024), 1)
    mov = ((row * 1024 + col) < 50000).astype(jnp.float32) * dt
    ox[...] = px[...] + gx * scale * mov
    oy[...] = py[...] + gy * scale * mov
    oz[...] = pz[...] + gz * scale * mov


def _tc_call(body, grid_rows, n_in, n_out, *args):
    spec = pl.BlockSpec((8, 1024), lambda i: (i, 0))
    return pl.pallas_call(
        body,
        grid=(grid_rows // 8,),
        in_specs=[spec] * n_in,
        out_specs=[spec] * n_out,
        out_shape=[jax.ShapeDtypeStruct((grid_rows, 1024), jnp.float32)] * n_out,
    )(*args)


def _padcol(a, n, npad, val=0):
    return jnp.pad(a, (0, npad - n), constant_values=val)


def kernel(pos, bond_idcs, bond_eq_val, bond_tolerance, angle_idcs,
           angle_eq_val, angle_tolerance, dih_idcs, dih_eq_val,
           movable_pos_idcs, dtau):
    f32 = jnp.float32
    px = jnp.pad(pos[:, 0], (0, NPOS - N_ATOM))
    py = jnp.pad(pos[:, 1], (0, NPOS - N_ATOM))
    pz = jnp.pad(pos[:, 2], (0, NPOS - N_ATOM))

    idx_flat = jnp.concatenate([
        _padcol(bond_idcs[:, 0], NB, NBP, DUMMY),
        _padcol(bond_idcs[:, 1], NB, NBP, DUMMY),
        _padcol(angle_idcs[:, 0], NA, NAP, DUMMY),
        _padcol(angle_idcs[:, 1], NA, NAP, DUMMY),
        _padcol(angle_idcs[:, 2], NA, NAP, DUMMY),
        _padcol(dih_idcs[:, 0], ND, NDP, DUMMY),
        _padcol(dih_idcs[:, 1], ND, NDP, DUMMY),
        _padcol(dih_idcs[:, 2], ND, NDP, DUMMY),
        _padcol(dih_idcs[:, 3], ND, NDP, DUMMY),
    ]).astype(jnp.int32)

    sc_gather, sc_scatter = _sc_kernels()
    xs, ys, zs = sc_gather(px, py, pz, idx_flat)
    comps = (xs, ys, zs)

    def plane(comp, off, cnt, rows):
        return lax.slice(comps[comp], (off,), (off + cnt,)).reshape(rows, 1024)

    ob0, ob1 = 0, NBP
    oa0, oa1, oa2 = 2 * NBP, 2 * NBP + NAP, 2 * NBP + 2 * NAP
    od0 = 2 * NBP + 3 * NAP
    od1, od2, od3 = od0 + NDP, od0 + 2 * NDP, od0 + 3 * NDP

    beq = _padcol(bond_eq_val, NB, NBP).reshape(104, 1024)
    btol = _padcol(bond_tolerance, NB, NBP).reshape(104, 1024)
    bond_in = [beq, btol]
    for off in (ob0, ob1):
        for c in range(3):
            bond_in.append(plane(c, off, NBP, 104))
    bond_out = _tc_call(_bond_body, 104, 8, 6, *bond_in)

    aeq = _padcol(angle_eq_val, NA, NAP).reshape(200, 1024)
    atol = _padcol(angle_tolerance, NA, NAP).reshape(200, 1024)
    angle_in = [aeq, atol]
    for off in (oa0, oa1, oa2):
        for c in range(3):
            angle_in.append(plane(c, off, NAP, 200))
    angle_out = _tc_call(_angle_body, 200, 11, 9, *angle_in)

    deq = _padcol(dih_eq_val, ND, NDP).reshape(296, 1024)
    dih_in = [deq]
    for off in (od0, od1, od2, od3):
        for c in range(3):
            dih_in.append(plane(c, off, NDP, 296))
    dih_out = _tc_call(_dih_body, 296, 13, 12, *dih_in)

    planes = []
    for c in range(3):
        planes.append(jnp.concatenate([
            bond_out[0 + c].ravel(), bond_out[3 + c].ravel(),
            angle_out[0 + c].ravel(), angle_out[3 + c].ravel(),
            angle_out[6 + c].ravel(),
            dih_out[0 + c].ravel(), dih_out[3 + c].ravel(),
            dih_out[6 + c].ravel(), dih_out[9 + c].ravel(),
        ]))

    zeros_stripe = jnp.zeros((STRIPE,), f32)
    gx2, gy2, gz2 = sc_scatter(planes[0], planes[1], planes[2],
                               idx_flat, zeros_stripe)

    outs = pl.pallas_call(
        _update_body,
        in_specs=[pl.BlockSpec(memory_space=pltpu.SMEM)] + [pl.BlockSpec()] * 9,
        out_shape=[jax.ShapeDtypeStruct((98, 1024), f32)] * 3,
    )(dtau,
      px.reshape(98, 1024), py.reshape(98, 1024), pz.reshape(98, 1024),
      gx2[:NPOS].reshape(98, 1024), gy2[:NPOS].reshape(98, 1024),
      gz2[:NPOS].reshape(98, 1024),
      gx2[NPOS:].reshape(98, 1024), gy2[NPOS:].reshape(98, 1024),
      gz2[NPOS:].reshape(98, 1024))
    new_pos = jnp.stack(outs).reshape(3, NPOS).T[:N_ATOM]
    return new_pos


# speedup vs baseline: 14.2741x; 1.0301x over previous
import functools

import jax
import jax.numpy as jnp
import numpy as np
from jax import lax
from jax.experimental import pallas as pl
from jax.experimental.pallas import tpu as pltpu
from jax.experimental.pallas import tpu_sc as plsc

NB, NA, ND = 100000, 200000, 300000
N_ATOM = 100000
NBP, NAP, NDP = 106496, 204800, 303104
NPOS = 100352
DUMMY = NPOS - 1
TOT = 2 * NBP + 3 * NAP + 4 * NDP
NW = 32
PER_W = TOT // NW
CHUNK = 3984
NCHUNK = PER_W // CHUNK
STRIPE = NPOS // 16

@functools.cache
def _sc_kernels():
    mesh = plsc.VectorSubcoreMesh(core_axis_name="c", subcore_axis_name="s")
    cparams = pltpu.CompilerParams(use_tc_tiling_on_sc=False)
    f32 = jnp.float32

    @functools.partial(
        pl.kernel, mesh=mesh, compiler_params=cparams,
        out_type=[jax.ShapeDtypeStruct((TOT,), f32)] * 3,
        scratch_types=(
            [pltpu.VMEM((CHUNK,), jnp.int32)] * 2
            + [pltpu.VMEM((CHUNK,), f32)] * 6
            + [pltpu.SemaphoreType.DMA] * 3
        ),
    )
    def sc_gather(px_hbm, py_hbm, pz_hbm, idx_hbm, ox_hbm, oy_hbm, oz_hbm,
                  idx0, idx1, xv0, xv1, yv0, yv1, zv0, zv1,
                  sem_idx, sem_gat, sem_st):
        wid = lax.axis_index("s") * 2 + lax.axis_index("c")
        base0 = wid * PER_W
        idxs = (idx0, idx1)
        xvs, yvs, zvs = (xv0, xv1), (yv0, yv1), (zv0, zv1)
        d_idx = [None, None]
        pend_st = [None, None]
        d_idx[0] = pltpu.async_copy(
            idx_hbm.at[pl.ds(base0, CHUNK)], idxs[0], sem_idx)
        for j in range(NCHUNK):
            b = j % 2
            sl = pl.ds(base0 + j * CHUNK, CHUNK)
            d_idx[b].wait()
            if pend_st[b] is not None:
                for d in pend_st[b]:
                    d.wait()
            gx = pltpu.async_copy(px_hbm.at[idxs[b]], xvs[b], sem_gat)
            gy = pltpu.async_copy(py_hbm.at[idxs[b]], yvs[b], sem_gat)
            gz = pltpu.async_copy(pz_hbm.at[idxs[b]], zvs[b], sem_gat)
            if j + 1 < NCHUNK:
                d_idx[1 - b] = pltpu.async_copy(
                    idx_hbm.at[pl.ds(base0 + (j + 1) * CHUNK, CHUNK)],
                    idxs[1 - b], sem_idx)
            gx.wait()
            gy.wait()
            gz.wait()
            pend_st[b] = [
                pltpu.async_copy(xvs[b], ox_hbm.at[sl], sem_st),
                pltpu.async_copy(yvs[b], oy_hbm.at[sl], sem_st),
                pltpu.async_copy(zvs[b], oz_hbm.at[sl], sem_st),
            ]
        for b in range(2):
            if pend_st[b] is not None:
                for d in pend_st[b]:
                    d.wait()

    @functools.partial(
        pl.kernel, mesh=mesh, compiler_params=cparams,
        out_type=[jax.ShapeDtypeStruct((2 * NPOS,), f32)] * 3,
        scratch_types=(
            [pltpu.VMEM((CHUNK,), jnp.int32)] * 2
            + [pltpu.VMEM((CHUNK,), f32)] * 6
            + [pltpu.SemaphoreType.DMA] * 2
            + [pltpu.VMEM_SHARED((NPOS,), f32)] * 3
        ),
    )
    def sc_scatter(cx_hbm, cy_hbm, cz_hbm, idx_hbm, zeros_hbm,
                   ox_hbm, oy_hbm, oz_hbm,
                   idx0, idx1, xv0, xv1, yv0, yv1, zv0, zv1,
                   sem_ld, sem_add, gx_sh, gy_sh, gz_sh):
        cid = lax.axis_index("c")
        sid = lax.axis_index("s")
        wid = sid * 2 + cid
        base0 = wid * PER_W
        stripe = pl.ds(sid * STRIPE, STRIPE)
        idxs = (idx0, idx1)
        xvs, yvs, zvs = (xv0, xv1), (yv0, yv1), (zv0, zv1)

        def fire_loads(j, b):
            sl = pl.ds(base0 + j * CHUNK, CHUNK)
            return [
                pltpu.async_copy(idx_hbm.at[sl], idxs[b], sem_ld),
                pltpu.async_copy(cx_hbm.at[sl], xvs[b], sem_ld),
                pltpu.async_copy(cy_hbm.at[sl], yvs[b], sem_ld),
                pltpu.async_copy(cz_hbm.at[sl], zvs[b], sem_ld),
            ]

        pend_ld = [None, None]
        pend_add = [None, None]
        pend_ld[0] = fire_loads(0, 0)
        pltpu.sync_copy(zeros_hbm, gx_sh.at[stripe])
        pltpu.sync_copy(zeros_hbm, gy_sh.at[stripe])
        pltpu.sync_copy(zeros_hbm, gz_sh.at[stripe])
        plsc.subcore_barrier()
        for j in range(NCHUNK):
            b = j % 2
            for d in pend_ld[b]:
                d.wait()
            pend_add[b] = [
                pltpu.async_copy(xvs[b], gx_sh.at[idxs[b]], sem_add, add=True),
                pltpu.async_copy(yvs[b], gy_sh.at[idxs[b]], sem_add, add=True),
                pltpu.async_copy(zvs[b], gz_sh.at[idxs[b]], sem_add, add=True),
            ]
            if j + 1 < NCHUNK:
                if pend_add[1 - b] is not None:
                    for d in pend_add[1 - b]:
                        d.wait()
                pend_ld[1 - b] = fire_loads(j + 1, 1 - b)
        for b in range(2):
            if pend_add[b] is not None:
                for d in pend_add[b]:
                    d.wait()
        plsc.subcore_barrier()
        out_off = pl.ds(cid * NPOS + sid * STRIPE, STRIPE)
        pltpu.sync_copy(gx_sh.at[stripe], ox_hbm.at[out_off])
        pltpu.sync_copy(gy_sh.at[stripe], oy_hbm.at[out_off])
        pltpu.sync_copy(gz_sh.at[stripe], oz_hbm.at[out_off])

    return sc_gather, sc_scatter


def _sin_poly(t):
    t2 = t * t
    return t * (1.0 + t2 * (-1.0 / 6 + t2 * (1.0 / 120 + t2 * (-1.0 / 5040 + t2 / 362880))))


def _cos_poly(t):
    t2 = t * t
    return 1.0 + t2 * (-0.5 + t2 * (1.0 / 24 + t2 * (-1.0 / 720 + t2 * (1.0 / 40320 - t2 / 3628800))))


def _arccos_poly(c):
    t = jnp.abs(c)
    s = jnp.sqrt(1.0 - t)
    p = 1.5707288 + t * (-0.2121144 + t * (0.0742610 - 0.0187293 * t))
    r = s * p
    return jnp.where(c >= 0, r, np.pi - r)


def _bond_body(eq, tol, x0, y0, z0, x1, y1, z1, ox0, oy0, oz0, ox1, oy1, oz1):
    vx = x1[...] - x0[...]
    vy = y1[...] - y0[...]
    vz = z1[...] - z0[...]
    r = jnp.sqrt(vx * vx + vy * vy + vz * vz)
    ir = 1.0 / r
    dr = r - eq[...]
    t = tol[...]
    act = ((dr * dr - t * t) > 0).astype(jnp.float32)
    coef = (2000.0 / NB) * dr * act * ir
    ox1[...] = coef * vx
    oy1[...] = coef * vy
    oz1[...] = coef * vz
    ox0[...] = -coef * vx
    oy0[...] = -coef * vy
    oz0[...] = -coef * vz


def _angle_body(eq, tol, x0, y0, z0, x1, y1, z1, x2, y2, z2,
                o0x, o0y, o0z, o1x, o1y, o1z, o2x, o2y, o2z):
    b0x = x0[...] - x1[...]
    b0y = y0[...] - y1[...]
    b0z = z0[...] - z1[...]
    b1x = x2[...] - x1[...]
    b1y = y2[...] - y1[...]
    b1z = z2[...] - z1[...]
    n0 = jnp.sqrt(b0x * b0x + b0y * b0y + b0z * b0z)
    n1 = jnp.sqrt(b1x * b1x + b1y * b1y + b1z * b1z)
    d = b0x * b1x + b0y * b1y + b0z * b1z
    q = n0 * n1 + 1e-12
    c = d / q
    lo, hi = -1.0 + 1e-7, 1.0 - 1e-7
    c_cl = jnp.clip(c, lo, hi)
    theta = _arccos_poly(c_cl)
    dth = theta - eq[...]
    t = tol[...]
    act = ((dth * dth - t * t) > 0).astype(jnp.float32)
    gtheta = (300.0 / NA) * dth * act
    inside = ((c > lo) & (c < hi)).astype(jnp.float32)
    gc = -gtheta * lax.rsqrt(1.0 - c_cl * c_cl) * inside
    gd = gc / q
    gq = -gc * d / (q * q)
    f0 = gq * n1 / n0
    f1 = gq * n0 / n1
    g0x = gd * b1x + f0 * b0x
    g0y = gd * b1y + f0 * b0y
    g0z = gd * b1z + f0 * b0z
    g1x = gd * b0x + f1 * b1x
    g1y = gd * b0y + f1 * b1y
    g1z = gd * b0z + f1 * b1z
    o0x[...] = g0x
    o0y[...] = g0y
    o0z[...] = g0z
    o2x[...] = g1x
    o2y[...] = g1y
    o2z[...] = g1z
    o1x[...] = -(g0x + g1x)
    o1y[...] = -(g0y + g1y)
    o1z[...] = -(g0z + g1z)


def _dih_body(eq, x0, y0, z0, x1, y1, z1, x2, y2, z2, x3, y3, z3,
              o0x, o0y, o0z, o1x, o1y, o1z, o2x, o2y, o2z, o3x, o3y, o3z):
    b0x = x0[...] - x1[...]
    b0y = y0[...] - y1[...]
    b0z = z0[...] - z1[...]
    b1x = x2[...] - x1[...]
    b1y = y2[...] - y1[...]
    b1z = z2[...] - z1[...]
    b2x = x3[...] - x2[...]
    b2y = y3[...] - y2[...]
    b2z = z3[...] - z2[...]
    n1 = jnp.sqrt(b1x * b1x + b1y * b1y + b1z * b1z)
    inb = 1.0 / (n1 + 1e-12)
    ux, uy, uz = b1x * inb, b1y * inb, b1z * inb
    sv = b0x * ux + b0y * uy + b0z * uz
    vx_, vy_, vz_ = b0x - sv * ux, b0y - sv * uy, b0z - sv * uz
    sw = b2x * ux + b2y * uy + b2z * uz
    wx, wy, wz = b2x - sw * ux, b2y - sw * uy, b2z - sw * uz
    crx = uy * vz_ - uz * vy_
    cry = uz * vx_ - ux * vz_
    crz = ux * vy_ - uy * vx_
    x = vx_ * wx + vy_ * wy + vz_ * wz
    y = crx * wx + cry * wy + crz * wz
    den = x * x + y * y
    iden = 1.0 / den
    irho = lax.rsqrt(den)
    sphi = y * irho
    cphi = x * irho
    e = eq[...]
    seq = _sin_poly(e)
    ceq = _cos_poly(e)
    sdlt = sphi * ceq - cphi * seq
    gphi = (2.0 / ND) * sdlt
    gx = -y * iden * gphi
    gy = x * iden * gphi
    gvx, gvy, gvz = gx * wx, gx * wy, gx * wz
    gwx = gx * vx_ + gy * crx
    gwy = gx * vy_ + gy * cry
    gwz = gx * vz_ + gy * crz
    gcrx, gcry, gcrz = gy * wx, gy * wy, gy * wz
    gux = vy_ * gcrz - vz_ * gcry
    guy = vz_ * gcrx - vx_ * gcrz
    guz = vx_ * gcry - vy_ * gcrx
    gvx += gcry * uz - gcrz * uy
    gvy += gcrz * ux - gcrx * uz
    gvz += gcrx * uy - gcry * ux
    gb2x, gb2y, gb2z = gwx, gwy, gwz
    gsw = -(gwx * ux + gwy * uy + gwz * uz)
    gux -= sw * gwx
    guy -= sw * gwy
    guz -= sw * gwz
    gb2x += gsw * ux
    gb2y += gsw * uy
    gb2z += gsw * uz
    gux += gsw * b2x
    guy += gsw * b2y
    guz += gsw * b2z
    gb0x, gb0y, gb0z = gvx, gvy, gvz
    gsv = -(gvx * ux + gvy * uy + gvz * uz)
    gux -= sv * gvx
    guy -= sv * gvy
    guz -= sv * gvz
    gb0x += gsv * ux
    gb0y += gsv * uy
    gb0z += gsv * uz
    gux += gsv * b0x
    guy += gsv * b0y
    guz += gsv * b0z
    gb1x = gux * inb
    gb1y = guy * inb
    gb1z = guz * inb
    gn1 = -(gux * b1x + guy * b1y + guz * b1z) * inb * inb
    fin = gn1 / n1
    gb1x += fin * b1x
    gb1y += fin * b1y
    gb1z += fin * b1z
    o0x[...] = gb0x
    o0y[...] = gb0y
    o0z[...] = gb0z
    o1x[...] = -gb0x - gb1x
    o1y[...] = -gb0y - gb1y
    o1z[...] = -gb0z - gb1z
    o2x[...] = gb1x - gb2x
    o2y[...] = gb1y - gb2y
    o2z[...] = gb1z - gb2z
    o3x[...] = gb2x
    o3y[...] = gb2y
    o3z[...] = gb2z


def _update_body(dt_ref, px, py, pz, ax, ay, az, bx, by, bz, ox, oy, oz):
    dt = dt_ref[0]

    def clean(v):
        return jnp.where(jnp.isnan(v), 0.0, v)

    gx = clean(-(ax[...] + bx[...]))
    gy = clean(-(ay[...] + by[...]))
    gz = clean(-(az[...] + bz[...]))
    fn = jnp.sqrt(gx * gx + gy * gy + gz * gz)
    thresh = 0.1 / dt
    scale = jnp.where(fn > thresh, thresh / (fn + 1e-12), 1.0)
    row = lax.broadcasted_iota(jnp.int32, (98, 1024), 0)
    col = lax.broadcasted_iota(jnp.int32, (98, 1024), 1)
    mov = ((row * 1024 + col) < 50000).astype(jnp.float32) * dt
    ox[...] = px[...] + gx * scale * mov
    oy[...] = py[...] + gy * scale * mov
    oz[...] = pz[...] + gz * scale * mov


def _tc_call(body, grid_rows, n_in, n_out, *args):
    spec = pl.BlockSpec((8, 1024), lambda i: (i, 0))
    return pl.pallas_call(
        body,
        grid=(grid_rows // 8,),
        in_specs=[spec] * n_in,
        out_specs=[spec] * n_out,
        out_shape=[jax.ShapeDtypeStruct((grid_rows, 1024), jnp.float32)] * n_out,
    )(*args)


def _padcol(a, n, npad, val=0):
    return jnp.pad(a, (0, npad - n), constant_values=val)


def kernel(pos, bond_idcs, bond_eq_val, bond_tolerance, angle_idcs,
           angle_eq_val, angle_tolerance, dih_idcs, dih_eq_val,
           movable_pos_idcs, dtau):
    f32 = jnp.float32
    px = jnp.pad(pos[:, 0], (0, NPOS - N_ATOM))
    py = jnp.pad(pos[:, 1], (0, NPOS - N_ATOM))
    pz = jnp.pad(pos[:, 2], (0, NPOS - N_ATOM))

    idx_flat = jnp.concatenate([
        _padcol(bond_idcs[:, 0], NB, NBP, DUMMY),
        _padcol(bond_idcs[:, 1], NB, NBP, DUMMY),
        _padcol(angle_idcs[:, 0], NA, NAP, DUMMY),
        _padcol(angle_idcs[:, 1], NA, NAP, DUMMY),
        _padcol(angle_idcs[:, 2], NA, NAP, DUMMY),
        _padcol(dih_idcs[:, 0], ND, NDP, DUMMY),
        _padcol(dih_idcs[:, 1], ND, NDP, DUMMY),
        _padcol(dih_idcs[:, 2], ND, NDP, DUMMY),
        _padcol(dih_idcs[:, 3], ND, NDP, DUMMY),
    ]).astype(jnp.int32)

    sc_gather, sc_scatter = _sc_kernels()
    xs, ys, zs = sc_gather(px, py, pz, idx_flat)
    comps = (xs, ys, zs)

    def plane(comp, off, cnt, rows):
        return lax.slice(comps[comp], (off,), (off + cnt,)).reshape(rows, 1024)

    ob0, ob1 = 0, NBP
    oa0, oa1, oa2 = 2 * NBP, 2 * NBP + NAP, 2 * NBP + 2 * NAP
    od0 = 2 * NBP + 3 * NAP
    od1, od2, od3 = od0 + NDP, od0 + 2 * NDP, od0 + 3 * NDP

    beq = _padcol(bond_eq_val, NB, NBP).reshape(104, 1024)
    btol = _padcol(bond_tolerance, NB, NBP).reshape(104, 1024)
    bond_in = [beq, btol]
    for off in (ob0, ob1):
        for c in range(3):
            bond_in.append(plane(c, off, NBP, 104))
    bond_out = _tc_call(_bond_body, 104, 8, 6, *bond_in)

    aeq = _padcol(angle_eq_val, NA, NAP).reshape(200, 1024)
    atol = _padcol(angle_tolerance, NA, NAP).reshape(200, 1024)
    angle_in = [aeq, atol]
    for off in (oa0, oa1, oa2):
        for c in range(3):
            angle_in.append(plane(c, off, NAP, 200))
    angle_out = _tc_call(_angle_body, 200, 11, 9, *angle_in)

    deq = _padcol(dih_eq_val, ND, NDP).reshape(296, 1024)
    dih_in = [deq]
    for off in (od0, od1, od2, od3):
        for c in range(3):
            dih_in.append(plane(c, off, NDP, 296))
    dih_out = _tc_call(_dih_body, 296, 13, 12, *dih_in)

    planes = []
    for c in range(3):
        planes.append(jnp.concatenate([
            bond_out[0 + c].ravel(), bond_out[3 + c].ravel(),
            angle_out[0 + c].ravel(), angle_out[3 + c].ravel(),
            angle_out[6 + c].ravel(),
            dih_out[0 + c].ravel(), dih_out[3 + c].ravel(),
            dih_out[6 + c].ravel(), dih_out[9 + c].ravel(),
        ]))

    zeros_stripe = jnp.zeros((STRIPE,), f32)
    gx2, gy2, gz2 = sc_scatter(planes[0], planes[1], planes[2],
                               idx_flat, zeros_stripe)

    outs = pl.pallas_call(
        _update_body,
        in_specs=[pl.BlockSpec(memory_space=pltpu.SMEM)] + [pl.BlockSpec()] * 9,
        out_shape=[jax.ShapeDtypeStruct((98, 1024), f32)] * 3,
    )(dtau,
      px.reshape(98, 1024), py.reshape(98, 1024), pz.reshape(98, 1024),
      gx2[:NPOS].reshape(98, 1024), gy2[:NPOS].reshape(98, 1024),
      gz2[:NPOS].reshape(98, 1024),
      gx2[NPOS:].reshape(98, 1024), gy2[NPOS:].reshape(98, 1024),
      gz2[NPOS:].reshape(98, 1024))
    new_pos = jnp.stack(outs).reshape(3, NPOS).T[:N_ATOM]
    return new_pos
